# baseline (speedup 1.0000x reference)
import functools

import jax
import jax.numpy as jnp
from jax import lax
from jax.experimental import pallas as pl
from jax.experimental.pallas import tpu as pltpu
from jax.experimental.pallas import tpu_sc as plsc

NC = 2
NS = 16
LANES = 16
NW = NC * NS
NBINS = 10240
KPAD = 40


def _wid():
    return lax.axis_index("s") * NC + lax.axis_index("c")


def _sc_mesh():
    return plsc.VectorSubcoreMesh(
        core_axis_name="c", subcore_axis_name="s",
        num_cores=NC, num_subcores=NS)


_SC_PARAMS = pltpu.CompilerParams(needs_layout_passes=False)


def _stats_body(K, RPW, nd_hbm, nb_hbm, idxout_hbm, pf_hbm, ff_hbm,
                rs_hbm, nd_v, nb_v, idx_v, f_v, hist_v, rs_v):
    DEG = K - 1
    wid = _wid()
    base = wid * RPW
    pltpu.sync_copy(nd_hbm.at[pl.ds(base, RPW)], nd_v)
    pltpu.sync_copy(nb_hbm.at[pl.ds(base * DEG, RPW * DEG)], nb_v)

    zeros16 = jnp.zeros((LANES,), jnp.float32)
    iota0 = lax.iota(jnp.int32, LANES)

    @pl.loop(0, NBINS // LANES, unroll=8)
    def _zero(i):
        hist_v[pl.ds(i * LANES, LANES)] = zeros16

    @pl.loop(0, RPW // LANES)
    def _transpose(g):
        rvec = iota0 + g * LANES
        for j in range(DEG):
            vj = plsc.load_gather(nb_v, [rvec * DEG + j])
            idx_v[pl.ds(j * RPW + g * LANES, LANES)] = vj
        idx_v[pl.ds(DEG * RPW + g * LANES, LANES)] = (
            nd_v[pl.ds(g * LANES, LANES)])

    @pl.loop(0, RPW // LANES)
    def _groups(g):
        base = g * LANES
        v = [idx_v[pl.ds(j * RPW + base, LANES)] for j in range(K)]
        rowcnt = zeros16
        for j in range(K):
            cnt = jnp.zeros((LANES,), jnp.int32)
            for k in range(j):
                cnt = cnt + jnp.where(v[j] == v[k], 1, 0).astype(jnp.int32)
            fj = jnp.where(cnt == 0, 1.0, 0.0).astype(jnp.float32)
            f_v[pl.ds(j * RPW + base, LANES)] = fj
            rowcnt = rowcnt + fj
        rs_v[pl.ds(base, LANES)] = rowcnt

    iota = lax.iota(jnp.int32, LANES)
    ones16 = jnp.ones((LANES,), jnp.float32)

    @pl.loop(0, RPW)
    def _histrows(r):
        rsplat = jnp.broadcast_to(jnp.int32(0) + r, (LANES,))
        for c in range(3):
            jv = iota + c * LANES
            jc = jnp.minimum(jv, K - 1)
            flat = jc * RPW + rsplat
            vals = plsc.load_gather(idx_v, [flat])
            fv = plsc.load_gather(f_v, [flat])
            m = jnp.logical_and(jv < K, fv > 0.5)
            plsc.addupdate_scatter(hist_v, [vals], ones16, mask=m)

    pltpu.sync_copy(idx_v, idxout_hbm.at[wid])
    pltpu.sync_copy(f_v, ff_hbm.at[wid])
    pltpu.sync_copy(rs_v, rs_hbm.at[wid])
    pltpu.sync_copy(hist_v, pf_hbm.at[wid])


def _norm_body(pf_ref, rs_ref, cs_ref, rsc_ref):
    cs = jnp.sum(pf_ref[...], axis=0, keepdims=True)
    cs_ref[...] = jnp.where(cs > 0.0, lax.rsqrt(cs), 1.0)
    rsc_ref[...] = lax.rsqrt(rs_ref[...])


def _gather_body(K, RPW, D, idx_hbm, ff_hbm, cs_hbm, rsc_hbm, tab_hbm,
                 out_hbm, idx_v, f_v, cs_v, rs_v, coef_v, lists_v, rows_v,
                 ob_v, sem0, sem1, sem2, osem):
    NCH = D // LANES
    RB = 8
    NBUF = 3
    wid = _wid()
    pltpu.sync_copy(idx_hbm.at[wid], idx_v)
    pltpu.sync_copy(ff_hbm.at[wid], f_v)
    pltpu.sync_copy(cs_hbm, cs_v)
    pltpu.sync_copy(rsc_hbm.at[wid], rs_v)

    iota = lax.iota(jnp.int32, LANES)

    @pl.loop(0, RPW)
    def _lists(r):
        rsplat = jnp.broadcast_to(jnp.int32(0) + r, (LANES,))
        for c in range((K + LANES - 1) // LANES):
            jv = iota + c * LANES
            jc = jnp.minimum(jv, K - 1)
            vals = plsc.load_gather(idx_v, [jc * RPW + rsplat])
            plsc.store_scatter(lists_v, [rsplat * K + jv], vals,
                               mask=jv < K)

    BLK = K * RB
    NBLK = RPW // RB
    sems = (sem0, sem1, sem2)

    def _fire(q, b):
        return pltpu.async_copy(
            tab_hbm.at[lists_v.at[pl.ds(q * BLK, BLK)]],
            rows_v.at[b], sems[b])

    for q in range(min(NBUF, NBLK)):
        _fire(q, q)

    @pl.loop(0, RPW // LANES)
    def _coef(g):
        base = g * LANES
        rsv = rs_v[pl.ds(base, LANES)]
        for j in range(K):
            vj = idx_v[pl.ds(j * RPW + base, LANES)]
            csg = plsc.load_gather(cs_v, [vj])
            coef_v[pl.ds(j * RPW + base, LANES)] = (
                f_v[pl.ds(j * RPW + base, LANES)] * csg * rsv)

    def _out_slice(q):
        return out_hbm.at[pl.ds(wid * RPW + q * RB, RB)]

    for q in range(NBLK):
        b = q % NBUF
        ob = q % 2
        pltpu.make_async_copy(
            tab_hbm.at[lists_v.at[pl.ds(q * BLK, BLK)]],
            rows_v.at[b], sems[b]).wait()
        if q >= 2:
            pltpu.make_async_copy(ob_v.at[ob], _out_slice(q - 2),
                                  osem).wait()

        @pl.loop(0, RB)
        def _rowloop(rb):
            r = q * RB + rb
            rsplat = jnp.broadcast_to(jnp.int32(0) + r, (LANES,))
            cvals = []
            for c in range((K + LANES - 1) // LANES):
                jv = iota + c * LANES
                jc = jnp.minimum(jv, K - 1)
                cvals.append(
                    plsc.load_gather(coef_v, [jc * RPW + rsplat]))
            acc = [jnp.zeros((LANES,), jnp.float32) for _ in range(NCH)]
            rbK = rb * K
            for j in range(K):
                s = cvals[j // LANES][j % LANES]
                for c in range(NCH):
                    acc[c] = acc[c] + s * rows_v[
                        b, rbK + j, pl.ds(c * LANES, LANES)]
            for c in range(NCH):
                ob_v[ob, rb, pl.ds(c * LANES, LANES)] = acc[c]

        if q + NBUF < NBLK:
            _fire(q + NBUF, b)
        pltpu.async_copy(ob_v.at[ob], _out_slice(q), osem)

    for q in range(max(NBLK - 2, 0), NBLK):
        pltpu.make_async_copy(ob_v.at[q % 2], _out_slice(q), osem).wait()


@functools.lru_cache(maxsize=None)
def _build(B, K, N, D):
    RPW = B // NW
    f32 = jnp.float32

    stats = pl.kernel(
        functools.partial(_stats_body, K, RPW),
        out_type=[
            jax.ShapeDtypeStruct((NW, K * RPW), jnp.int32),
            jax.ShapeDtypeStruct((NW, NBINS), f32),
            jax.ShapeDtypeStruct((NW, K * RPW), f32),
            jax.ShapeDtypeStruct((NW, RPW), f32),
        ],
        mesh=_sc_mesh(),
        compiler_params=_SC_PARAMS,
        scratch_types=[
            pltpu.VMEM((RPW,), jnp.int32),
            pltpu.VMEM((RPW * (K - 1),), jnp.int32),
            pltpu.VMEM((K * RPW,), jnp.int32),
            pltpu.VMEM((K * RPW,), f32),
            pltpu.VMEM((NBINS,), f32),
            pltpu.VMEM((RPW,), f32),
        ],
    )

    norm = pl.pallas_call(
        _norm_body,
        out_shape=[
            jax.ShapeDtypeStruct((1, NBINS), f32),
            jax.ShapeDtypeStruct((NW, RPW), f32),
        ],
    )

    gather = pl.kernel(
        functools.partial(_gather_body, K, RPW, D),
        out_type=jax.ShapeDtypeStruct((B, D), f32),
        mesh=_sc_mesh(),
        compiler_params=_SC_PARAMS,
        scratch_types=[
            pltpu.VMEM((K * RPW,), jnp.int32),
            pltpu.VMEM((K * RPW,), f32),
            pltpu.VMEM((NBINS,), f32),
            pltpu.VMEM((RPW,), f32),
            pltpu.VMEM((K * RPW,), f32),
            pltpu.VMEM((K * RPW,), jnp.int32),
            pltpu.VMEM((3, K * 8, D), f32),
            pltpu.VMEM((2, 8, D), f32),
            pltpu.SemaphoreType.DMA,
            pltpu.SemaphoreType.DMA,
            pltpu.SemaphoreType.DMA,
            pltpu.SemaphoreType.DMA,
        ],
    )
    return stats, norm, gather


def kernel(nodes, neighs, table):
    B, DEG = neighs.shape
    K = DEG + 1
    N, D = table.shape
    stats, norm, gather = _build(B, K, N, D)

    idx_blocks, partials, fflags, rowsum = stats(
        nodes, neighs.reshape(B * DEG))
    colscale2, rowscale = norm(partials, rowsum)
    colscale = colscale2.reshape(NBINS)
    out = gather(idx_blocks, fflags, colscale, rowscale, table)
    return out


# speedup vs baseline: 1.1230x; 1.1230x over previous
import functools

import jax
import jax.numpy as jnp
from jax import lax
from jax.experimental import pallas as pl
from jax.experimental.pallas import tpu as pltpu
from jax.experimental.pallas import tpu_sc as plsc

NC = 2
NS = 16
LANES = 16
NW = NC * NS
NBINS = 10240
KPAD = 40


def _wid():
    return lax.axis_index("s") * NC + lax.axis_index("c")


def _sc_mesh():
    return plsc.VectorSubcoreMesh(
        core_axis_name="c", subcore_axis_name="s",
        num_cores=NC, num_subcores=NS)


_SC_PARAMS = pltpu.CompilerParams(needs_layout_passes=False)


def _stats_body(K, RPW, nd_hbm, nb_hbm, idxout_hbm, pf_hbm, ff_hbm,
                rs_hbm, nd_v, nb_v, idx_v, f_v, hist_v, rs_v):
    DEG = K - 1
    wid = _wid()
    base = wid * RPW
    pltpu.sync_copy(nd_hbm.at[pl.ds(base, RPW)], nd_v)
    pltpu.sync_copy(nb_hbm.at[pl.ds(base * DEG, RPW * DEG)], nb_v)

    zeros16 = jnp.zeros((LANES,), jnp.float32)
    iota0 = lax.iota(jnp.int32, LANES)

    @pl.loop(0, NBINS // LANES, unroll=8)
    def _zero(i):
        hist_v[pl.ds(i * LANES, LANES)] = zeros16

    @pl.loop(0, RPW // LANES)
    def _transpose(g):
        rvec = iota0 + g * LANES
        for j in range(DEG):
            vj = plsc.load_gather(nb_v, [rvec * DEG + j])
            idx_v[pl.ds(j * RPW + g * LANES, LANES)] = vj
        idx_v[pl.ds(DEG * RPW + g * LANES, LANES)] = (
            nd_v[pl.ds(g * LANES, LANES)])

    @pl.loop(0, RPW // LANES)
    def _groups(g):
        base = g * LANES
        v = [idx_v[pl.ds(j * RPW + base, LANES)] for j in range(K)]
        rowcnt = zeros16
        for j in range(K):
            cnt = jnp.zeros((LANES,), jnp.int32)
            for k in range(j):
                cnt = cnt + jnp.where(v[j] == v[k], 1, 0).astype(jnp.int32)
            fj = jnp.where(cnt == 0, 1.0, 0.0).astype(jnp.float32)
            f_v[pl.ds(j * RPW + base, LANES)] = fj
            rowcnt = rowcnt + fj
        rs_v[pl.ds(base, LANES)] = rowcnt

    iota = lax.iota(jnp.int32, LANES)
    ones16 = jnp.ones((LANES,), jnp.float32)

    @pl.loop(0, RPW)
    def _histrows(r):
        rsplat = jnp.broadcast_to(jnp.int32(0) + r, (LANES,))
        for c in range(3):
            jv = iota + c * LANES
            jc = jnp.minimum(jv, K - 1)
            flat = jc * RPW + rsplat
            vals = plsc.load_gather(idx_v, [flat])
            fv = plsc.load_gather(f_v, [flat])
            m = jnp.logical_and(jv < K, fv > 0.5)
            plsc.addupdate_scatter(hist_v, [vals], ones16, mask=m)

    pltpu.sync_copy(idx_v, idxout_hbm.at[wid])
    pltpu.sync_copy(f_v, ff_hbm.at[wid])
    pltpu.sync_copy(rs_v, rs_hbm.at[wid])
    pltpu.sync_copy(hist_v, pf_hbm.at[wid])


def _norm_body(pf_ref, rs_ref, cs_ref, rsc_ref):
    cs = jnp.sum(pf_ref[...], axis=0, keepdims=True)
    cs_ref[...] = jnp.where(cs > 0.0, lax.rsqrt(cs), 1.0)
    rsc_ref[...] = lax.rsqrt(rs_ref[...])


def _gather_body(K, RPW, D, idx_hbm, ff_hbm, cs_hbm, rsc_hbm, tab_hbm,
                 out_hbm, idx_v, f_v, cs_v, rs_v, coef_v, lists_v, rows_v,
                 oblk_v, sem0, sem1):
    NCH = D // LANES
    RB = 8
    wid = _wid()
    ins = ((idx_hbm.at[wid], idx_v), (ff_hbm.at[wid], f_v),
           (cs_hbm, cs_v), (rsc_hbm.at[wid], rs_v))
    for src, dst in ins:
        pltpu.async_copy(src, dst, sem0)
    for src, dst in ins:
        pltpu.make_async_copy(src, dst, sem0).wait()

    iota = lax.iota(jnp.int32, LANES)

    @pl.loop(0, RPW)
    def _lists(r):
        rsplat = jnp.broadcast_to(jnp.int32(0) + r, (LANES,))
        for c in range((K + LANES - 1) // LANES):
            jv = iota + c * LANES
            jc = jnp.minimum(jv, K - 1)
            vals = plsc.load_gather(idx_v, [jc * RPW + rsplat])
            plsc.store_scatter(lists_v, [rsplat * K + jv], vals,
                               mask=jv < K)

    BLK = K * RB
    NBLK = RPW // RB
    sems = (sem0, sem1)

    def _fire(q, b):
        return pltpu.async_copy(
            tab_hbm.at[lists_v.at[pl.ds(q * BLK, BLK)]],
            rows_v.at[b], sems[b])

    _fire(0, 0)
    _fire(1, 1)

    @pl.loop(0, RPW // LANES)
    def _coef(g):
        base = g * LANES
        rsv = rs_v[pl.ds(base, LANES)]
        for j in range(K):
            vj = idx_v[pl.ds(j * RPW + base, LANES)]
            csg = plsc.load_gather(cs_v, [vj])
            coef_v[pl.ds(j * RPW + base, LANES)] = (
                f_v[pl.ds(j * RPW + base, LANES)] * csg * rsv)

    @pl.loop(0, NBLK, step=2)
    def _blocks(q0):
        for b in range(2):
            q = q0 + b
            pltpu.make_async_copy(
                tab_hbm.at[lists_v.at[pl.ds(q * BLK, BLK)]],
                rows_v.at[b], sems[b]).wait()

            @pl.loop(0, RB)
            def _rowloop(rb):
                r = q * RB + rb
                rsplat = jnp.broadcast_to(jnp.int32(0) + r, (LANES,))
                cvals = []
                for c in range((K + LANES - 1) // LANES):
                    jv = iota + c * LANES
                    jc = jnp.minimum(jv, K - 1)
                    cvals.append(
                        plsc.load_gather(coef_v, [jc * RPW + rsplat]))
                acc = [jnp.zeros((LANES,), jnp.float32)
                       for _ in range(NCH)]
                rbK = rb * K
                for j in range(K):
                    s = cvals[j // LANES][j % LANES]
                    for c in range(NCH):
                        acc[c] = acc[c] + s * rows_v[
                            b, rbK + j, pl.ds(c * LANES, LANES)]
                for c in range(NCH):
                    oblk_v[r, pl.ds(c * LANES, LANES)] = acc[c]

            @pl.when(q + 2 < NBLK)
            def _next():
                _fire(q + 2, b)

    pltpu.sync_copy(oblk_v, out_hbm.at[pl.ds(wid * RPW, RPW)])


@functools.lru_cache(maxsize=None)
def _build(B, K, N, D):
    RPW = B // NW
    f32 = jnp.float32

    stats = pl.kernel(
        functools.partial(_stats_body, K, RPW),
        out_type=[
            jax.ShapeDtypeStruct((NW, K * RPW), jnp.int32),
            jax.ShapeDtypeStruct((NW, NBINS), f32),
            jax.ShapeDtypeStruct((NW, K * RPW), f32),
            jax.ShapeDtypeStruct((NW, RPW), f32),
        ],
        mesh=_sc_mesh(),
        compiler_params=_SC_PARAMS,
        scratch_types=[
            pltpu.VMEM((RPW,), jnp.int32),
            pltpu.VMEM((RPW * (K - 1),), jnp.int32),
            pltpu.VMEM((K * RPW,), jnp.int32),
            pltpu.VMEM((K * RPW,), f32),
            pltpu.VMEM((NBINS,), f32),
            pltpu.VMEM((RPW,), f32),
        ],
    )

    norm = pl.pallas_call(
        _norm_body,
        out_shape=[
            jax.ShapeDtypeStruct((1, NBINS), f32),
            jax.ShapeDtypeStruct((NW, RPW), f32),
        ],
    )

    gather = pl.kernel(
        functools.partial(_gather_body, K, RPW, D),
        out_type=jax.ShapeDtypeStruct((B, D), f32),
        mesh=_sc_mesh(),
        compiler_params=_SC_PARAMS,
        scratch_types=[
            pltpu.VMEM((K * RPW,), jnp.int32),
            pltpu.VMEM((K * RPW,), f32),
            pltpu.VMEM((NBINS,), f32),
            pltpu.VMEM((RPW,), f32),
            pltpu.VMEM((K * RPW,), f32),
            pltpu.VMEM((K * RPW,), jnp.int32),
            pltpu.VMEM((2, K * 8, D), f32),
            pltpu.VMEM((RPW, D), f32),
            pltpu.SemaphoreType.DMA,
            pltpu.SemaphoreType.DMA,
        ],
    )
    return stats, norm, gather


def kernel(nodes, neighs, table):
    B, DEG = neighs.shape
    K = DEG + 1
    N, D = table.shape
    stats, norm, gather = _build(B, K, N, D)

    idx_blocks, partials, fflags, rowsum = stats(
        nodes, neighs.reshape(B * DEG))
    colscale2, rowscale = norm(partials, rowsum)
    colscale = colscale2.reshape(NBINS)
    out = gather(idx_blocks, fflags, colscale, rowscale, table)
    return out


# speedup vs baseline: 1.1244x; 1.0012x over previous
import functools

import jax
import jax.numpy as jnp
from jax import lax
from jax.experimental import pallas as pl
from jax.experimental.pallas import tpu as pltpu
from jax.experimental.pallas import tpu_sc as plsc

NC = 2
NS = 16
LANES = 16
NW = NC * NS
NBINS = 10240
KPAD = 40


def _wid():
    return lax.axis_index("s") * NC + lax.axis_index("c")


def _sc_mesh():
    return plsc.VectorSubcoreMesh(
        core_axis_name="c", subcore_axis_name="s",
        num_cores=NC, num_subcores=NS)


_SC_PARAMS = pltpu.CompilerParams(needs_layout_passes=False)


def _stats_body(K, RPW, nd_hbm, nb_hbm, idxout_hbm, pf_hbm, ff_hbm,
                rs_hbm, nd_v, nb_v, idx_v, f_v, hist_v, rs_v):
    DEG = K - 1
    wid = _wid()
    base = wid * RPW
    pltpu.sync_copy(nd_hbm.at[pl.ds(base, RPW)], nd_v)
    pltpu.sync_copy(nb_hbm.at[pl.ds(base * DEG, RPW * DEG)], nb_v)

    zeros16 = jnp.zeros((LANES,), jnp.float32)
    iota0 = lax.iota(jnp.int32, LANES)

    @pl.loop(0, NBINS // LANES, unroll=8)
    def _zero(i):
        hist_v[pl.ds(i * LANES, LANES)] = zeros16

    @pl.loop(0, RPW // LANES)
    def _transpose(g):
        rvec = iota0 + g * LANES
        for j in range(DEG):
            vj = plsc.load_gather(nb_v, [rvec * DEG + j])
            idx_v[pl.ds(j * RPW + g * LANES, LANES)] = vj
        idx_v[pl.ds(DEG * RPW + g * LANES, LANES)] = (
            nd_v[pl.ds(g * LANES, LANES)])

    @pl.loop(0, RPW // LANES)
    def _groups(g):
        base = g * LANES
        v = [idx_v[pl.ds(j * RPW + base, LANES)] for j in range(K)]
        rowcnt = zeros16
        for j in range(K):
            cnt = jnp.zeros((LANES,), jnp.int32)
            for k in range(j):
                cnt = cnt + jnp.where(v[j] == v[k], 1, 0).astype(jnp.int32)
            fj = jnp.where(cnt == 0, 1.0, 0.0).astype(jnp.float32)
            f_v[pl.ds(j * RPW + base, LANES)] = fj
            rowcnt = rowcnt + fj
        rs_v[pl.ds(base, LANES)] = rowcnt

    iota = lax.iota(jnp.int32, LANES)
    ones16 = jnp.ones((LANES,), jnp.float32)

    @pl.loop(0, RPW, unroll=2)
    def _histrows(r):
        rsplat = jnp.broadcast_to(jnp.int32(0) + r, (LANES,))
        for c in range(3):
            jv = iota + c * LANES
            jc = jnp.minimum(jv, K - 1)
            flat = jc * RPW + rsplat
            vals = plsc.load_gather(idx_v, [flat])
            fv = plsc.load_gather(f_v, [flat])
            m = jnp.logical_and(jv < K, fv > 0.5)
            plsc.addupdate_scatter(hist_v, [vals], ones16, mask=m)

    pltpu.sync_copy(idx_v, idxout_hbm.at[wid])
    pltpu.sync_copy(f_v, ff_hbm.at[wid])
    pltpu.sync_copy(rs_v, rs_hbm.at[wid])
    pltpu.sync_copy(hist_v, pf_hbm.at[wid])


def _norm_body(pf_ref, rs_ref, cs_ref, rsc_ref):
    cs = jnp.sum(pf_ref[...], axis=0, keepdims=True)
    cs_ref[...] = jnp.where(cs > 0.0, lax.rsqrt(cs), 1.0)
    rsc_ref[...] = lax.rsqrt(rs_ref[...])


def _gather_body(K, RPW, D, idx_hbm, ff_hbm, cs_hbm, rsc_hbm, tab_hbm,
                 out_hbm, idx_v, f_v, cs_v, rs_v, coef_v, lists_v, rows_v,
                 oblk_v, sem0, sem1):
    NCH = D // LANES
    RB = 8
    wid = _wid()
    ins = ((idx_hbm.at[wid], idx_v), (ff_hbm.at[wid], f_v),
           (cs_hbm, cs_v), (rsc_hbm.at[wid], rs_v))
    for src, dst in ins:
        pltpu.async_copy(src, dst, sem0)
    for src, dst in ins:
        pltpu.make_async_copy(src, dst, sem0).wait()

    iota = lax.iota(jnp.int32, LANES)

    @pl.loop(0, RPW, unroll=2)
    def _lists(r):
        rsplat = jnp.broadcast_to(jnp.int32(0) + r, (LANES,))
        for c in range((K + LANES - 1) // LANES):
            jv = iota + c * LANES
            jc = jnp.minimum(jv, K - 1)
            vals = plsc.load_gather(idx_v, [jc * RPW + rsplat])
            plsc.store_scatter(lists_v, [rsplat * K + jv], vals,
                               mask=jv < K)

    BLK = K * RB
    NBLK = RPW // RB
    sems = (sem0, sem1)

    def _fire(q, b):
        return pltpu.async_copy(
            tab_hbm.at[lists_v.at[pl.ds(q * BLK, BLK)]],
            rows_v.at[b], sems[b])

    _fire(0, 0)
    _fire(1, 1)

    @pl.loop(0, RPW // LANES)
    def _coef(g):
        base = g * LANES
        rsv = rs_v[pl.ds(base, LANES)]
        for j in range(K):
            vj = idx_v[pl.ds(j * RPW + base, LANES)]
            csg = plsc.load_gather(cs_v, [vj])
            coef_v[pl.ds(j * RPW + base, LANES)] = (
                f_v[pl.ds(j * RPW + base, LANES)] * csg * rsv)

    @pl.loop(0, NBLK, step=2)
    def _blocks(q0):
        for b in range(2):
            q = q0 + b
            pltpu.make_async_copy(
                tab_hbm.at[lists_v.at[pl.ds(q * BLK, BLK)]],
                rows_v.at[b], sems[b]).wait()

            @pl.loop(0, RB)
            def _rowloop(rb):
                r = q * RB + rb
                rsplat = jnp.broadcast_to(jnp.int32(0) + r, (LANES,))
                cvals = []
                for c in range((K + LANES - 1) // LANES):
                    jv = iota + c * LANES
                    jc = jnp.minimum(jv, K - 1)
                    cvals.append(
                        plsc.load_gather(coef_v, [jc * RPW + rsplat]))
                acc = [jnp.zeros((LANES,), jnp.float32)
                       for _ in range(NCH)]
                rbK = rb * K
                for j in range(K):
                    s = cvals[j // LANES][j % LANES]
                    for c in range(NCH):
                        acc[c] = acc[c] + s * rows_v[
                            b, rbK + j, pl.ds(c * LANES, LANES)]
                for c in range(NCH):
                    oblk_v[r, pl.ds(c * LANES, LANES)] = acc[c]

            @pl.when(q + 2 < NBLK)
            def _next():
                _fire(q + 2, b)

    pltpu.sync_copy(oblk_v, out_hbm.at[pl.ds(wid * RPW, RPW)])


@functools.lru_cache(maxsize=None)
def _build(B, K, N, D):
    RPW = B // NW
    f32 = jnp.float32

    stats = pl.kernel(
        functools.partial(_stats_body, K, RPW),
        out_type=[
            jax.ShapeDtypeStruct((NW, K * RPW), jnp.int32),
            jax.ShapeDtypeStruct((NW, NBINS), f32),
            jax.ShapeDtypeStruct((NW, K * RPW), f32),
            jax.ShapeDtypeStruct((NW, RPW), f32),
        ],
        mesh=_sc_mesh(),
        compiler_params=_SC_PARAMS,
        scratch_types=[
            pltpu.VMEM((RPW,), jnp.int32),
            pltpu.VMEM((RPW * (K - 1),), jnp.int32),
            pltpu.VMEM((K * RPW,), jnp.int32),
            pltpu.VMEM((K * RPW,), f32),
            pltpu.VMEM((NBINS,), f32),
            pltpu.VMEM((RPW,), f32),
        ],
    )

    norm = pl.pallas_call(
        _norm_body,
        out_shape=[
            jax.ShapeDtypeStruct((1, NBINS), f32),
            jax.ShapeDtypeStruct((NW, RPW), f32),
        ],
    )

    gather = pl.kernel(
        functools.partial(_gather_body, K, RPW, D),
        out_type=jax.ShapeDtypeStruct((B, D), f32),
        mesh=_sc_mesh(),
        compiler_params=_SC_PARAMS,
        scratch_types=[
            pltpu.VMEM((K * RPW,), jnp.int32),
            pltpu.VMEM((K * RPW,), f32),
            pltpu.VMEM((NBINS,), f32),
            pltpu.VMEM((RPW,), f32),
            pltpu.VMEM((K * RPW,), f32),
            pltpu.VMEM((K * RPW,), jnp.int32),
            pltpu.VMEM((2, K * 8, D), f32),
            pltpu.VMEM((RPW, D), f32),
            pltpu.SemaphoreType.DMA,
            pltpu.SemaphoreType.DMA,
        ],
    )
    return stats, norm, gather


def kernel(nodes, neighs, table):
    B, DEG = neighs.shape
    K = DEG + 1
    N, D = table.shape
    stats, norm, gather = _build(B, K, N, D)

    idx_blocks, partials, fflags, rowsum = stats(
        nodes, neighs.reshape(B * DEG))
    colscale2, rowscale = norm(partials, rowsum)
    colscale = colscale2.reshape(NBINS)
    out = gather(idx_blocks, fflags, colscale, rowscale, table)
    return out
